# baseline (speedup 1.0000x reference)
import functools

import jax
import jax.numpy as jnp
from jax import lax
from jax.experimental import pallas as pl
from jax.experimental.pallas import tpu as pltpu
from jax.experimental.pallas import tpu_sc as plsc

_NUM_ENT = 100000
_HIDDEN = 16
_BATCH = 1024

_info = plsc.get_sparse_core_info()
_NC, _NS = _info.num_cores, _info.num_subcores
_NW = _NC * _NS
_BPW = _BATCH // _NW

_sc_mesh = plsc.VectorSubcoreMesh(core_axis_name="c", subcore_axis_name="s")


@functools.partial(
    pl.kernel,
    mesh=_sc_mesh,
    out_type=jax.ShapeDtypeStruct((_BATCH, _HIDDEN + 1), jnp.float32),
    scratch_types=[
        pltpu.VMEM((_BPW,), jnp.int32),
        pltpu.VMEM((_BPW, _HIDDEN, 128), jnp.float32),
        pltpu.VMEM((_BPW, _HIDDEN + 1), jnp.float32),
        pltpu.SemaphoreType.DMA,
    ],
    compiler_params=pltpu.CompilerParams(needs_layout_passes=False),
)
def _sc_gather(table_hbm, idx_hbm, out_hbm, idx_v, blk_v, rows_v, sem):
    wid = lax.axis_index("s") * _NC + lax.axis_index("c")
    base = wid * _BPW
    pltpu.sync_copy(idx_hbm.at[pl.ds(base, _BPW)], idx_v)
    lanes = lax.iota(jnp.int32, 16)
    zeros = jnp.zeros((16,), jnp.int32)
    es = []
    copies = []
    for g in range(_BPW // 16):
        v = idx_v[pl.ds(g * 16, 16)]
        for m in range(16):
            e = jnp.sum(jnp.where(lanes == m, v, zeros))
            es.append(e)
            eb = pl.multiple_of((e // 128) * 128, 128)
            copies.append(pltpu.async_copy(
                table_hbm.at[:, pl.ds(eb, 128)],
                blk_v.at[g * 16 + m],
                sem,
            ))
    for j, cp in enumerate(copies):
        cp.wait()
        lane = jnp.full((16,), 1, jnp.int32) * (es[j] % 128)
        val = plsc.load_gather(blk_v.at[j], [lanes, lane])
        plsc.store_scatter(rows_v, [jnp.full((16,), j, jnp.int32), lanes], val)
    ones = jnp.ones((16,), jnp.float32)
    col16 = jnp.full((16,), _HIDDEN, jnp.int32)
    for g in range(_BPW // 16):
        plsc.store_scatter(rows_v, [g * 16 + lanes, col16], ones)
    pltpu.sync_copy(rows_v, out_hbm.at[pl.ds(base, _BPW)])


_KA = _HIDDEN + 1
_BN = 6272
_NT = (_NUM_ENT + _BN - 1) // _BN


def _mm_body(wa_ref, ea_ref, out_ref):
    out_ref[...] = lax.dot_general(
        wa_ref[...], ea_ref[...],
        (((0,), (1,)), ((), ())),
        preferred_element_type=jnp.float32,
    )


def kernel(batch_data, ent_emb, W, b):
    idx = batch_data.reshape(_BATCH).astype(jnp.int32)
    ea = _sc_gather(ent_emb.T, idx)
    wa = jnp.concatenate([W.T, b.reshape(1, _NUM_ENT)], axis=0)
    out_t = pl.pallas_call(
        _mm_body,
        grid=(_NT,),
        in_specs=[
            pl.BlockSpec((_KA, _BN), lambda i: (0, i)),
            pl.BlockSpec((_BATCH, _KA), lambda i: (0, 0)),
        ],
        out_specs=pl.BlockSpec((_BN, _BATCH), lambda i: (i, 0)),
        out_shape=jax.ShapeDtypeStruct((_NUM_ENT, _BATCH), jnp.float32),
    )(wa, ea)
    return out_t.T


# speedup vs baseline: 1.0157x; 1.0157x over previous
import functools

import jax
import jax.numpy as jnp
from jax import lax
from jax.experimental import pallas as pl
from jax.experimental.pallas import tpu as pltpu
from jax.experimental.pallas import tpu_sc as plsc

_NUM_ENT = 100000
_HIDDEN = 16
_BATCH = 1024

_info = plsc.get_sparse_core_info()
_NC, _NS = _info.num_cores, _info.num_subcores
_NW = _NC * _NS
_BPW = _BATCH // _NW

_sc_mesh = plsc.VectorSubcoreMesh(core_axis_name="c", subcore_axis_name="s")


@functools.partial(
    pl.kernel,
    mesh=_sc_mesh,
    out_type=jax.ShapeDtypeStruct((_BATCH, _HIDDEN + 1), jnp.float32),
    scratch_types=[
        pltpu.VMEM((_BPW,), jnp.int32),
        pltpu.VMEM((_BPW, _HIDDEN, 128), jnp.float32),
        pltpu.VMEM((_BPW, _HIDDEN + 1), jnp.float32),
        pltpu.SemaphoreType.DMA,
    ],
    compiler_params=pltpu.CompilerParams(needs_layout_passes=False),
)
def _sc_gather(table_hbm, idx_hbm, out_hbm, idx_v, blk_v, rows_v, sem):
    wid = lax.axis_index("s") * _NC + lax.axis_index("c")
    base = wid * _BPW
    pltpu.sync_copy(idx_hbm.at[pl.ds(base, _BPW)], idx_v)
    lanes = lax.iota(jnp.int32, 16)
    zeros = jnp.zeros((16,), jnp.int32)
    es = []
    copies = []
    for g in range(_BPW // 16):
        v = idx_v[pl.ds(g * 16, 16)]
        for m in range(16):
            e = jnp.sum(jnp.where(lanes == m, v, zeros))
            es.append(e)
            eb = pl.multiple_of((e // 128) * 128, 128)
            copies.append(pltpu.async_copy(
                table_hbm.at[:, pl.ds(eb, 128)],
                blk_v.at[g * 16 + m],
                sem,
            ))
    for j, cp in enumerate(copies):
        cp.wait()
        lane = jnp.full((16,), 1, jnp.int32) * (es[j] % 128)
        val = plsc.load_gather(blk_v.at[j], [lanes, lane])
        plsc.store_scatter(rows_v, [jnp.full((16,), j, jnp.int32), lanes], val)
    ones = jnp.ones((16,), jnp.float32)
    col16 = jnp.full((16,), _HIDDEN, jnp.int32)
    for g in range(_BPW // 16):
        plsc.store_scatter(rows_v, [g * 16 + lanes, col16], ones)
    pltpu.sync_copy(rows_v, out_hbm.at[pl.ds(base, _BPW)])


_KA = _HIDDEN + 1
_BN = 2048
_NT = (_NUM_ENT + _BN - 1) // _BN


def _mm_body(wa_ref, ea_ref, out_ref):
    out_ref[...] = lax.dot_general(
        wa_ref[...], ea_ref[...],
        (((0,), (1,)), ((), ())),
        preferred_element_type=jnp.float32,
    )


def kernel(batch_data, ent_emb, W, b):
    idx = batch_data.reshape(_BATCH).astype(jnp.int32)
    ea = _sc_gather(ent_emb.T, idx)
    wa = jnp.concatenate([W.T, b.reshape(1, _NUM_ENT)], axis=0)
    out_t = pl.pallas_call(
        _mm_body,
        grid=(_NT,),
        in_specs=[
            pl.BlockSpec((_KA, _BN), lambda i: (0, i)),
            pl.BlockSpec((_BATCH, _KA), lambda i: (0, 0)),
        ],
        out_specs=pl.BlockSpec((_BN, _BATCH), lambda i: (i, 0)),
        out_shape=jax.ShapeDtypeStruct((_NUM_ENT, _BATCH), jnp.float32),
    )(wa, ea)
    return out_t.T
